# Initial kernel scaffold; baseline (speedup 1.0000x reference)
import jax
import jax.numpy as jnp
from jax.experimental import pallas as pl


def kernel(z_t, W_in, b_in, in_proj_W, conv_w, conv_b, x_proj_W, dt_proj_W, dt_proj_b, A_log, D_param, out_proj_W, router_W, eW1, eb1, eW2, eb2, mn_g, mn_b, on_g, on_b, W_out, b_out, ln_g, ln_b):
    raise NotImplementedError("write your pallas kernel here")


# speedup vs baseline: 27.9313x; 27.9313x over previous
import functools

import jax
import jax.numpy as jnp
from jax.experimental import pallas as pl
from jax.experimental.pallas import tpu as pltpu

F32 = jnp.float32

_D_IN = 1024
_D_MODEL = 256
_D_OUT = 64
_NL = 2
_E = 8
_DS = 16
_DC = 4
_DI = 512
_DTR = 16
_DFF = 1024
_T = 2048

_TB = 256
_TC = 128


def _ln(x, g, b, eps=1e-5):
    m = jnp.mean(x, axis=-1, keepdims=True)
    v = jnp.mean((x - m) * (x - m), axis=-1, keepdims=True)
    return (x - m) * jax.lax.rsqrt(v + eps) * g + b


def _silu(x):
    return x * jax.nn.sigmoid(x)


def _dot(a, b):
    return jnp.dot(a, b, preferred_element_type=F32)


def _kin_body(z_ref, w_ref, b_ref, o_ref):
    o_ref[...] = _dot(z_ref[...], w_ref[...]) + b_ref[...]


def _input_proj(z2d, W_in, b_in):
    nb = _T // _TB
    return pl.pallas_call(
        _kin_body,
        grid=(nb,),
        in_specs=[
            pl.BlockSpec((_TB, _D_IN), lambda i: (i, 0)),
            pl.BlockSpec((_D_IN, _D_MODEL), lambda i: (0, 0)),
            pl.BlockSpec((1, _D_MODEL), lambda i: (0, 0)),
        ],
        out_specs=pl.BlockSpec((_TB, _D_MODEL), lambda i: (i, 0)),
        out_shape=jax.ShapeDtypeStruct((_T, _D_MODEL), F32),
    )(z2d, W_in, b_in.reshape(1, -1))


def _kfront_body(x_ref, g_ref, b_ref, wi_ref, cw_ref, cb_ref, wx_ref,
                 wdt_ref, bdt_ref, xc_ref, z_ref, dt_ref, bc_ref, carry_ref):
    i = pl.program_id(0)

    @pl.when(i == 0)
    def _():
        carry_ref[...] = jnp.zeros_like(carry_ref)

    xn = _ln(x_ref[...], g_ref[...], b_ref[...])
    xz = _dot(xn, wi_ref[...])
    xs = xz[:, :_DI]
    zb = xz[:, _DI:]
    carry = carry_ref[0:_DC - 1]
    seg = jnp.concatenate([carry, xs], axis=0)
    conv = cb_ref[...]
    for j in range(_DC):
        conv = conv + seg[j:j + _TB] * cw_ref[j:j + 1, :]
    carry_ref[0:_DC - 1] = xs[_TB - (_DC - 1):_TB]
    xc = _silu(conv)
    xc_ref[...] = xc
    z_ref[...] = zb
    xdbl = _dot(xc, wx_ref[...])
    bc_ref[...] = xdbl[:, _DTR:]
    dt_ref[...] = jax.nn.softplus(_dot(xdbl[:, :_DTR], wdt_ref[...])
                                  + bdt_ref[...])


def _mamba_front(x, mg, mb, Wi, cwT, cb, Wx, Wdt, bdt):
    nb = _T // _TB
    outs = (
        jax.ShapeDtypeStruct((_T, _DI), F32),
        jax.ShapeDtypeStruct((_T, _DI), F32),
        jax.ShapeDtypeStruct((_T, _DI), F32),
        jax.ShapeDtypeStruct((_T, 2 * _DS), F32),
    )
    return pl.pallas_call(
        _kfront_body,
        grid=(nb,),
        in_specs=[
            pl.BlockSpec((_TB, _D_MODEL), lambda i: (i, 0)),
            pl.BlockSpec((1, _D_MODEL), lambda i: (0, 0)),
            pl.BlockSpec((1, _D_MODEL), lambda i: (0, 0)),
            pl.BlockSpec((_D_MODEL, 2 * _DI), lambda i: (0, 0)),
            pl.BlockSpec((_DC, _DI), lambda i: (0, 0)),
            pl.BlockSpec((1, _DI), lambda i: (0, 0)),
            pl.BlockSpec((_DI, _DTR + 2 * _DS), lambda i: (0, 0)),
            pl.BlockSpec((_DTR, _DI), lambda i: (0, 0)),
            pl.BlockSpec((1, _DI), lambda i: (0, 0)),
        ],
        out_specs=(
            pl.BlockSpec((_TB, _DI), lambda i: (i, 0)),
            pl.BlockSpec((_TB, _DI), lambda i: (i, 0)),
            pl.BlockSpec((_TB, _DI), lambda i: (i, 0)),
            pl.BlockSpec((_TB, 2 * _DS), lambda i: (i, 0)),
        ),
        out_shape=outs,
        scratch_shapes=[pltpu.VMEM((8, _DI), F32)],
    )(x, mg.reshape(1, -1), mb.reshape(1, -1), Wi, cwT, cb.reshape(1, -1),
      Wx, Wdt, bdt.reshape(1, -1))


def _kscan_body(xc_ref, z_ref, dt_ref, bc_ref, negA_ref, dp_ref, wo_ref,
                r_ref, o_ref, dA_ref, dH_ref, h_ref):
    i = pl.program_id(0)

    @pl.when(i == 0)
    def _():
        h_ref[...] = jnp.zeros_like(h_ref)

    dt_c = dt_ref[...]
    xc_c = xc_ref[...]
    u = dt_c * xc_c
    B_c = bc_ref[:, :_DS]
    C_c = bc_ref[:, _DS:]
    dA_ref[...] = jnp.exp(dt_c[:, None, :] * negA_ref[...][None, :, :])
    dH_ref[...] = B_c[:, :, None] * u[:, None, :]

    def body(t, h):
        hn = dA_ref[t] * h + dH_ref[t]
        dH_ref[t] = hn
        return hn

    h = jax.lax.fori_loop(0, _TC, body, h_ref[...])
    h_ref[...] = h
    Y = jnp.sum(dH_ref[...] * C_c[:, :, None], axis=1)
    y = (Y + dp_ref[...] * xc_c) * _silu(z_ref[...])
    o_ref[...] = r_ref[...] + _dot(y, wo_ref[...])


def _mamba_scan(xc, z, dt, bc, negA_T, Dp, Wo, r):
    nb = _T // _TC
    return pl.pallas_call(
        _kscan_body,
        grid=(nb,),
        in_specs=[
            pl.BlockSpec((_TC, _DI), lambda i: (i, 0)),
            pl.BlockSpec((_TC, _DI), lambda i: (i, 0)),
            pl.BlockSpec((_TC, _DI), lambda i: (i, 0)),
            pl.BlockSpec((_TC, 2 * _DS), lambda i: (i, 0)),
            pl.BlockSpec((_DS, _DI), lambda i: (0, 0)),
            pl.BlockSpec((1, _DI), lambda i: (0, 0)),
            pl.BlockSpec((_DI, _D_MODEL), lambda i: (0, 0)),
            pl.BlockSpec((_TC, _D_MODEL), lambda i: (i, 0)),
        ],
        out_specs=pl.BlockSpec((_TC, _D_MODEL), lambda i: (i, 0)),
        out_shape=jax.ShapeDtypeStruct((_T, _D_MODEL), F32),
        scratch_shapes=[
            pltpu.VMEM((_TC, _DS, _DI), F32),
            pltpu.VMEM((_TC, _DS, _DI), F32),
            pltpu.VMEM((_DS, _DI), F32),
        ],
    )(xc, z, dt, bc, negA_T, Dp.reshape(1, -1), Wo, r)


def _krouter_body(x_ref, g_ref, b_ref, wr_ref, xn_ref, gates_ref):
    xn = _ln(x_ref[...], g_ref[...], b_ref[...])
    xn_ref[...] = xn
    logits = _dot(xn, wr_ref[...])
    mx = jnp.max(logits, axis=-1, keepdims=True)
    ex = jnp.exp(logits - mx)
    probs = ex / jnp.sum(ex, axis=-1, keepdims=True)
    lane = jax.lax.broadcasted_iota(jnp.int32, probs.shape, 1)
    i1 = jnp.argmax(probs, axis=-1)[:, None]
    p1 = jnp.max(probs, axis=-1, keepdims=True)
    masked = jnp.where(lane == i1, -jnp.inf, probs)
    i2 = jnp.argmax(masked, axis=-1)[:, None]
    p2 = jnp.max(masked, axis=-1, keepdims=True)
    denom = p1 + p2
    gates_ref[...] = (jnp.where(lane == i1, p1, 0.0)
                      + jnp.where(lane == i2, p2, 0.0)) / denom


def _router(x, og, ob, Wr):
    nb = _T // _TB
    return pl.pallas_call(
        _krouter_body,
        grid=(nb,),
        in_specs=[
            pl.BlockSpec((_TB, _D_MODEL), lambda i: (i, 0)),
            pl.BlockSpec((1, _D_MODEL), lambda i: (0, 0)),
            pl.BlockSpec((1, _D_MODEL), lambda i: (0, 0)),
            pl.BlockSpec((_D_MODEL, _E), lambda i: (0, 0)),
        ],
        out_specs=(
            pl.BlockSpec((_TB, _D_MODEL), lambda i: (i, 0)),
            pl.BlockSpec((_TB, _E), lambda i: (i, 0)),
        ),
        out_shape=(
            jax.ShapeDtypeStruct((_T, _D_MODEL), F32),
            jax.ShapeDtypeStruct((_T, _E), F32),
        ),
    )(x, og.reshape(1, -1), ob.reshape(1, -1), Wr)


def _kffn_body(xn_ref, gates_ref, r_ref, w1_ref, b1_ref, w2_ref, b2_ref,
               o_ref):
    e = pl.program_id(1)

    @pl.when(e == 0)
    def _():
        o_ref[...] = r_ref[...]

    hpre = _dot(xn_ref[...], w1_ref[0]) + b1_ref[0]
    h = 0.5 * hpre * (1.0 + jax.lax.erf(hpre * 0.7071067811865476))
    o = _dot(h, w2_ref[0]) + b2_ref[0]
    g = gates_ref[...]
    lane = jax.lax.broadcasted_iota(jnp.int32, g.shape, 1)
    ge = jnp.sum(jnp.where(lane == e, g, 0.0), axis=-1, keepdims=True)
    o_ref[...] += ge * o


def _moe_ffn(xn, gates, r, W1, b1, W2, b2):
    nb = _T // _TB
    return pl.pallas_call(
        _kffn_body,
        grid=(nb, _E),
        in_specs=[
            pl.BlockSpec((_TB, _D_MODEL), lambda i, e: (i, 0)),
            pl.BlockSpec((_TB, _E), lambda i, e: (i, 0)),
            pl.BlockSpec((_TB, _D_MODEL), lambda i, e: (i, 0)),
            pl.BlockSpec((1, _D_MODEL, _DFF), lambda i, e: (e, 0, 0)),
            pl.BlockSpec((1, 1, _DFF), lambda i, e: (e, 0, 0)),
            pl.BlockSpec((1, _DFF, _D_MODEL), lambda i, e: (e, 0, 0)),
            pl.BlockSpec((1, 1, _D_MODEL), lambda i, e: (e, 0, 0)),
        ],
        out_specs=pl.BlockSpec((_TB, _D_MODEL), lambda i, e: (i, 0)),
        out_shape=jax.ShapeDtypeStruct((_T, _D_MODEL), F32),
    )(xn, gates, r, W1, b1.reshape(_E, 1, _DFF), W2,
      b2.reshape(_E, 1, _D_MODEL))


def _kout_body(x_ref, w_ref, b_ref, g_ref, bb_ref, o_ref):
    o = _dot(x_ref[...], w_ref[...]) + b_ref[...]
    o_ref[...] = _ln(o, g_ref[...], bb_ref[...])


def _out_proj(x, W_out, b_out, ln_g, ln_b):
    nb = _T // _TB
    return pl.pallas_call(
        _kout_body,
        grid=(nb,),
        in_specs=[
            pl.BlockSpec((_TB, _D_MODEL), lambda i: (i, 0)),
            pl.BlockSpec((_D_MODEL, _D_OUT), lambda i: (0, 0)),
            pl.BlockSpec((1, _D_OUT), lambda i: (0, 0)),
            pl.BlockSpec((1, _D_OUT), lambda i: (0, 0)),
            pl.BlockSpec((1, _D_OUT), lambda i: (0, 0)),
        ],
        out_specs=pl.BlockSpec((_TB, _D_OUT), lambda i: (i, 0)),
        out_shape=jax.ShapeDtypeStruct((_T, _D_OUT), F32),
    )(x, W_out, b_out.reshape(1, -1), ln_g.reshape(1, -1),
      ln_b.reshape(1, -1))


def kernel(z_t, W_in, b_in, in_proj_W, conv_w, conv_b, x_proj_W, dt_proj_W,
           dt_proj_b, A_log, D_param, out_proj_W, router_W, eW1, eb1, eW2,
           eb2, mn_g, mn_b, on_g, on_b, W_out, b_out, ln_g, ln_b):
    z2d = z_t.reshape(_T, _D_IN)
    x = _input_proj(z2d, W_in, b_in)
    for i in range(_NL):
        cwT = jnp.transpose(conv_w[i], (1, 0))
        negA_T = -jnp.exp(jnp.transpose(A_log[i], (1, 0)))
        xc, z, dt, bc = _mamba_front(x, mn_g[i], mn_b[i], in_proj_W[i], cwT,
                                     conv_b[i], x_proj_W[i], dt_proj_W[i],
                                     dt_proj_b[i])
        x = _mamba_scan(xc, z, dt, bc, negA_T, D_param[i], out_proj_W[i], x)
        xn, gates = _router(x, on_g[i], on_b[i], router_W[i])
        x = _moe_ffn(xn, gates, x, eW1[i], eb1[i], eW2[i], eb2[i])
    out = _out_proj(x, W_out, b_out, ln_g, ln_b)
    return out.reshape(1, _T, _D_OUT)
